# baseline (speedup 1.0000x reference)
import functools

import jax
import jax.numpy as jnp
from jax import lax
from jax.experimental import pallas as pl
from jax.experimental.pallas import tpu as pltpu
from jax.experimental.pallas import tpu_sc as plsc

NUM_FACTORS = 32
BATCH = 16384
NC = 2
NS = 16
NW = NC * NS
BPW = BATCH // NW
CHUNK = 128
NCH = BPW // CHUNK

_mesh = plsc.VectorSubcoreMesh(
    core_axis_name="c", subcore_axis_name="s", num_cores=NC, num_subcores=NS
)


@functools.partial(
    pl.kernel,
    out_type=jax.ShapeDtypeStruct((BATCH,), jnp.float32),
    mesh=_mesh,
    compiler_params=pltpu.CompilerParams(
        needs_layout_passes=False, use_tc_tiling_on_sc=False),
    scratch_types=[
        pltpu.VMEM((NCH, CHUNK), jnp.int32),
        pltpu.VMEM((NCH, CHUNK), jnp.int32),
        pltpu.VMEM((BPW, NUM_FACTORS), jnp.float32),
        pltpu.VMEM((BPW, NUM_FACTORS), jnp.float32),
        pltpu.VMEM((BPW,), jnp.float32),
        pltpu.SemaphoreType.DMA,
    ],
)
def _mf_dot(uf_hbm, if_hbm, user_hbm, item_hbm, out_hbm,
            uidx_v, iidx_v, urows_v, irows_v, out_v, sem):
    wid = lax.axis_index("s") * NC + lax.axis_index("c")
    base = wid * BPW

    pltpu.sync_copy(user_hbm.at[wid], uidx_v)
    pltpu.sync_copy(item_hbm.at[wid], iidx_v)

    copies = []
    for j in range(NCH):
        copies.append(pltpu.async_copy(
            uf_hbm.at[uidx_v.at[j]], urows_v.at[pl.ds(j * CHUNK, CHUNK)], sem))
        copies.append(pltpu.async_copy(
            if_hbm.at[iidx_v.at[j]], irows_v.at[pl.ds(j * CHUNK, CHUNK)], sem))
    for cp in copies:
        cp.wait()

    iota16 = lax.iota(jnp.int32, 16)

    def block(bb, _):
        rows = bb * 16 + iota16
        acc = jnp.zeros((16,), jnp.float32)
        for f in range(NUM_FACTORS):
            fv = jnp.full((16,), f, jnp.int32)
            u = plsc.load_gather(urows_v, [rows, fv])
            it = plsc.load_gather(irows_v, [rows, fv])
            acc = acc + u * it
        out_v[pl.ds(pl.multiple_of(bb * 16, 16), 16)] = acc
        return 0

    lax.fori_loop(0, BPW // 16, block, 0)

    pltpu.sync_copy(out_v, out_hbm.at[pl.ds(base, BPW)])


def kernel(user_factors, item_factors, user, item):
    user_c = user.astype(jnp.int32).reshape(NW, NCH, CHUNK)
    item_c = item.astype(jnp.int32).reshape(NW, NCH, CHUNK)
    return _mf_dot(user_factors, item_factors, user_c, item_c)
